# baseline (speedup 1.0000x reference)
import functools

import jax
import jax.numpy as jnp
from jax import lax
from jax.experimental import pallas as pl
from jax.experimental.pallas import tpu as pltpu
from jax.experimental.pallas import tpu_sc as plsc


@functools.lru_cache(maxsize=None)
def _build(B, S, D, V, NC, NS):
    NW = NC * NS
    BS = B * S
    assert BS % NW == 0 and S % (BS // NW) == 0 and D % 16 == 0
    b_per_w = BS // NW
    D2 = 2 * D
    mesh = plsc.VectorSubcoreMesh(core_axis_name="c", subcore_axis_name="s")

    @functools.partial(
        pl.kernel,
        mesh=mesh,
        out_type=jax.ShapeDtypeStruct((B, D, S), jnp.float32),
        compiler_params=pltpu.CompilerParams(needs_layout_passes=False),
        scratch_types=[
            pltpu.VMEM((b_per_w,), jnp.int32),
            pltpu.VMEM((b_per_w,), jnp.int32),
            pltpu.VMEM((b_per_w,), jnp.int32),
            pltpu.VMEM((b_per_w, D2), jnp.float32),
            pltpu.VMEM((D, b_per_w), jnp.float32),
            pltpu.VMEM((D, b_per_w), jnp.float32),
            pltpu.SemaphoreType.DMA,
        ],
    )
    def emb_kernel(idx_hbm, x_hbm, table_hbm, posT_hbm, out_hbm,
                   idx_v, x_v, half_v, rows_v, pos_v, outT_v, sem):
        wid = lax.axis_index("s") * NC + lax.axis_index("c")
        base = wid * b_per_w
        b = lax.div(base, S)
        s0 = lax.rem(base, S)
        pltpu.sync_copy(idx_hbm.at[pl.ds(base, b_per_w)], idx_v)
        gather = pltpu.async_copy(table_hbm.at[idx_v], rows_v, sem)
        pltpu.sync_copy(x_hbm.at[pl.ds(base, b_per_w)], x_v)
        pltpu.sync_copy(posT_hbm.at[:, pl.ds(s0, b_per_w)], pos_v)

        def par_body(t, carry):
            x = x_v[pl.ds(t * 16, 16)]
            half_v[pl.ds(t * 16, 16)] = (x & 1) * D
            return carry

        lax.fori_loop(0, b_per_w // 16, par_body, 0)
        gather.wait()

        lanes = lax.iota(jnp.int32, 16)

        def repack_d(d, d_splat):
            for t in range(b_per_w // 16):
                sl = pl.ds(t * 16, 16)
                rows = lanes + (t * 16)
                cols = half_v[sl] + d_splat
                val = plsc.load_gather(rows_v, [rows, cols])
                outT_v[d, sl] = val + pos_v[d, sl]
            return d_splat + 1

        lax.fori_loop(0, D, repack_d, jnp.zeros((16,), jnp.int32))
        pltpu.sync_copy(outT_v, out_hbm.at[b, :, pl.ds(s0, b_per_w)])

    return emb_kernel


def kernel(X, tok_embd, pos_embd):
    B, S = X.shape
    V, D = tok_embd.shape
    try:
        info = plsc.get_sparse_core_info()
        NC, NS = info.num_cores, info.num_subcores
    except Exception:
        NC, NS = 2, 16
    xf = X.reshape(B * S).astype(jnp.int32)
    fn = _build(B, S, D, V, NC, NS)
    out_bds = fn(xf >> 1, xf, tok_embd.reshape(V // 2, 2 * D), pos_embd.T)
    return jnp.swapaxes(out_bds, 1, 2)


# speedup vs baseline: 1.1252x; 1.1252x over previous
import functools

import jax
import jax.numpy as jnp
from jax import lax
from jax.experimental import pallas as pl
from jax.experimental.pallas import tpu as pltpu
from jax.experimental.pallas import tpu_sc as plsc


@functools.lru_cache(maxsize=None)
def _build(B, S, D, V, NC, NS):
    NW = NC * NS
    BS = B * S
    assert BS % NW == 0 and S % (BS // NW) == 0 and D % 16 == 0
    b_per_w = BS // NW
    DP = 128
    mesh = plsc.VectorSubcoreMesh(core_axis_name="c", subcore_axis_name="s")

    @functools.partial(
        pl.kernel,
        mesh=mesh,
        out_type=jax.ShapeDtypeStruct((B, D, S), jnp.float32),
        compiler_params=pltpu.CompilerParams(needs_layout_passes=False),
        scratch_types=[
            pltpu.VMEM((b_per_w,), jnp.int32),
            pltpu.VMEM((b_per_w, DP), jnp.float32),
            pltpu.VMEM((D, b_per_w), jnp.float32),
            pltpu.VMEM((D, b_per_w), jnp.float32),
            pltpu.SemaphoreType.DMA,
        ],
    )
    def emb_kernel(x_hbm, table_hbm, posT_hbm, out_hbm,
                   x_v, rows_v, pos_v, outT_v, sem):
        wid = lax.axis_index("s") * NC + lax.axis_index("c")
        base = wid * b_per_w
        b = lax.div(base, S)
        s0 = lax.rem(base, S)
        pltpu.sync_copy(x_hbm.at[pl.ds(base, b_per_w)], x_v)
        gather = pltpu.async_copy(table_hbm.at[x_v], rows_v, sem)
        pltpu.sync_copy(posT_hbm.at[:, pl.ds(s0, b_per_w)], pos_v)
        gather.wait()

        lanes = lax.iota(jnp.int32, 16)

        def repack_d(d, d_splat):
            for t in range(b_per_w // 16):
                sl = pl.ds(t * 16, 16)
                rows = lanes + (t * 16)
                val = plsc.load_gather(rows_v, [rows, d_splat])
                outT_v[d, sl] = val + pos_v[d, sl]
            return d_splat + 1

        lax.fori_loop(0, D, repack_d, jnp.zeros((16,), jnp.int32))
        pltpu.sync_copy(outT_v, out_hbm.at[b, :, pl.ds(s0, b_per_w)])

    return emb_kernel


def kernel(X, tok_embd, pos_embd):
    B, S = X.shape
    V, D = tok_embd.shape
    try:
        info = plsc.get_sparse_core_info()
        NC, NS = info.num_cores, info.num_subcores
    except Exception:
        NC, NS = 2, 16
    fn = _build(B, S, D, V, NC, NS)
    table_p = jnp.pad(tok_embd, ((0, 0), (0, 128 - D)))
    out_bds = fn(X.reshape(B * S).astype(jnp.int32), table_p, pos_embd.T)
    return jnp.swapaxes(out_bds, 1, 2)
